# Initial kernel scaffold; baseline (speedup 1.0000x reference)
import functools

import jax
import jax.numpy as jnp
from jax import lax
from jax.experimental import pallas as pl
from jax.experimental.pallas import tpu as pltpu
from jax.experimental.pallas import tpu_sc as plsc

NUM_EMB = 1000000
EMB = 32
HID = 64
B = 4096
T = 50

NC = 2
NS = 16
NW = NC * NS

N_ROWS = B * T
BPW = N_ROWS // NW
CHUNK = 1600
NCHUNK = BPW // CHUNK


@functools.partial(
    pl.kernel,
    out_type=jax.ShapeDtypeStruct((N_ROWS, EMB), jnp.float32),
    mesh=plsc.VectorSubcoreMesh(core_axis_name="c", subcore_axis_name="s"),
    scratch_types=[
        pltpu.VMEM((BPW,), jnp.int32),
        pltpu.VMEM((2, CHUNK, EMB), jnp.float32),
        pltpu.SemaphoreType.DMA,
        pltpu.SemaphoreType.DMA,
    ],
)
def _gather(idx_hbm, table_hbm, out_hbm, idx_v, rows_v, gsem, osem):
    wid = lax.axis_index("s") * NC + lax.axis_index("c")
    base = wid * BPW
    pltpu.sync_copy(idx_hbm.at[pl.ds(base, BPW)], idx_v)

    def gather_chunk(i, slot):
        return pltpu.async_copy(
            table_hbm.at[idx_v.at[pl.ds(i * CHUNK, CHUNK)]],
            rows_v.at[slot],
            gsem,
        )

    gather_chunk(0, 0).wait()
    for i in range(NCHUNK):
        if i + 1 < NCHUNK:
            nxt = gather_chunk(i + 1, (i + 1) % 2)
        out_cp = pltpu.async_copy(
            rows_v.at[i % 2],
            out_hbm.at[pl.ds(base + i * CHUNK, CHUNK)],
            osem,
        )
        if i + 1 < NCHUNK:
            nxt.wait()
        out_cp.wait()


BT = 1024


def _gru_body(x_ref, wih_ref, whh_ref, bih_ref, bhh_ref, out_ref, h_ref):
    t = pl.program_id(1)

    @pl.when(t == 0)
    def _():
        h_ref[...] = jnp.zeros_like(h_ref)

    x = x_ref[:, 0, :]
    h = h_ref[...]
    gi = jnp.dot(x, wih_ref[...], preferred_element_type=jnp.float32) + bih_ref[...]
    gh = jnp.dot(h, whh_ref[...], preferred_element_type=jnp.float32) + bhh_ref[...]
    r = jax.nn.sigmoid(gi[:, :HID] + gh[:, :HID])
    z = jax.nn.sigmoid(gi[:, HID:2 * HID] + gh[:, HID:2 * HID])
    n = jnp.tanh(gi[:, 2 * HID:] + r * gh[:, 2 * HID:])
    h_new = (1.0 - z) * n + z * h
    h_ref[...] = h_new
    out_ref[:, 0, :] = h_new


def _gru(embedded, wih_t, whh_t, bih, bhh, interpret=False):
    return pl.pallas_call(
        _gru_body,
        grid=(B // BT, T),
        in_specs=[
            pl.BlockSpec((BT, 1, EMB), lambda b, t: (b, t, 0)),
            pl.BlockSpec((EMB, 3 * HID), lambda b, t: (0, 0)),
            pl.BlockSpec((HID, 3 * HID), lambda b, t: (0, 0)),
            pl.BlockSpec((1, 3 * HID), lambda b, t: (0, 0)),
            pl.BlockSpec((1, 3 * HID), lambda b, t: (0, 0)),
        ],
        out_specs=pl.BlockSpec((BT, 1, HID), lambda b, t: (b, t, 0)),
        out_shape=jax.ShapeDtypeStruct((B, T, HID), jnp.float32),
        scratch_shapes=[pltpu.VMEM((BT, HID), jnp.float32)],
        interpret=interpret,
    )(embedded, wih_t, whh_t, bih, bhh)


@jax.jit
def kernel(input, table, W_ih, W_hh, b_ih, b_hh):
    flat_idx = input.reshape(-1)
    emb_flat = _gather(flat_idx, table)
    embedded = emb_flat.reshape(B, T, EMB)
    out = _gru(embedded, W_ih.T, W_hh.T, b_ih[None], b_hh[None])
    h_n = out[:, T - 1, :][None]
    return out, h_n


# speedup vs baseline: 6.5494x; 6.5494x over previous
import functools

import jax
import jax.numpy as jnp
from jax import lax
from jax.experimental import pallas as pl
from jax.experimental.pallas import tpu as pltpu
from jax.experimental.pallas import tpu_sc as plsc

NUM_EMB = 1000000
EMB = 32
HID = 64
B = 4096
T = 50

NC = 2
NS = 16
NW = NC * NS

N_ROWS = B * T
BPW = N_ROWS // NW
CHUNK = 1600
NCHUNK = BPW // CHUNK


@functools.cache
def _make_gather():
    @functools.partial(
        pl.kernel,
        out_type=jax.ShapeDtypeStruct((N_ROWS, EMB), jnp.float32),
        mesh=plsc.VectorSubcoreMesh(core_axis_name="c", subcore_axis_name="s"),
        compiler_params=pltpu.CompilerParams(use_tc_tiling_on_sc=False),
        scratch_types=[
            pltpu.VMEM((BPW,), jnp.int32),
            pltpu.VMEM((2, CHUNK, EMB), jnp.float32),
            pltpu.SemaphoreType.DMA,
            pltpu.SemaphoreType.DMA,
        ],
    )
    def _gather(idx_hbm, table_hbm, out_hbm, idx_v, rows_v, gsem, osem):
        wid = lax.axis_index("s") * NC + lax.axis_index("c")
        base = wid * BPW
        pltpu.sync_copy(idx_hbm.at[pl.ds(base, BPW)], idx_v)

        def gather_chunk(i, slot):
            return pltpu.async_copy(
                table_hbm.at[idx_v.at[pl.ds(i * CHUNK, CHUNK)]],
                rows_v.at[slot],
                gsem,
            )

        gather_chunk(0, 0).wait()
        for i in range(NCHUNK):
            if i + 1 < NCHUNK:
                nxt = gather_chunk(i + 1, (i + 1) % 2)
            out_cp = pltpu.async_copy(
                rows_v.at[i % 2],
                out_hbm.at[pl.ds(base + i * CHUNK, CHUNK)],
                osem,
            )
            if i + 1 < NCHUNK:
                nxt.wait()
            out_cp.wait()

    return _gather


BT = 1024


def _gru_body(x_ref, wih_ref, whh_ref, bih_ref, bhh_ref, out_ref, h_ref):
    t = pl.program_id(1)

    @pl.when(t == 0)
    def _():
        h_ref[...] = jnp.zeros_like(h_ref)

    x = x_ref[...]
    h = h_ref[...]
    gi = jnp.dot(x, wih_ref[...], preferred_element_type=jnp.float32) + bih_ref[...]
    gh = jnp.dot(h, whh_ref[...], preferred_element_type=jnp.float32) + bhh_ref[...]
    r = jax.nn.sigmoid(gi[:, :HID] + gh[:, :HID])
    z = jax.nn.sigmoid(gi[:, HID:2 * HID] + gh[:, HID:2 * HID])
    n = jnp.tanh(gi[:, 2 * HID:] + r * gh[:, 2 * HID:])
    h_new = (1.0 - z) * n + z * h
    h_ref[...] = h_new
    out_ref[...] = h_new


NB = B // BT


def _gru(emb2d, wih_t, whh_t, bih, bhh, interpret=False):
    return pl.pallas_call(
        _gru_body,
        grid=(NB, T),
        in_specs=[
            pl.BlockSpec((BT, EMB), lambda b, t: (t * NB + b, 0)),
            pl.BlockSpec((EMB, 3 * HID), lambda b, t: (0, 0)),
            pl.BlockSpec((HID, 3 * HID), lambda b, t: (0, 0)),
            pl.BlockSpec((1, 3 * HID), lambda b, t: (0, 0)),
            pl.BlockSpec((1, 3 * HID), lambda b, t: (0, 0)),
        ],
        out_specs=pl.BlockSpec((BT, HID), lambda b, t: (t * NB + b, 0)),
        out_shape=jax.ShapeDtypeStruct((T * B, HID), jnp.float32),
        scratch_shapes=[pltpu.VMEM((BT, HID), jnp.float32)],
        interpret=interpret,
    )(emb2d, wih_t, whh_t, bih, bhh)


@jax.jit
def kernel(input, table, W_ih, W_hh, b_ih, b_hh):
    flat_idx = input.T.reshape(-1)
    emb2d = _make_gather()(flat_idx, table)
    out2d = _gru(emb2d, W_ih.T, W_hh.T, b_ih[None], b_hh[None])
    out = jnp.swapaxes(out2d.reshape(T, B, HID), 0, 1)
    h_n = out2d[(T - 1) * B:, :][None]
    return out, h_n


# speedup vs baseline: 7.4223x; 1.1333x over previous
import functools

import jax
import jax.numpy as jnp
from jax import lax
from jax.experimental import pallas as pl
from jax.experimental.pallas import tpu as pltpu
from jax.experimental.pallas import tpu_sc as plsc

NUM_EMB = 1000000
EMB = 32
HID = 64
B = 4096
T = 50

NC = 2
NS = 16
NW = NC * NS

N_ROWS = B * T
BPW = N_ROWS // NW
CHUNK = 1600
NCHUNK = BPW // CHUNK


@functools.cache
def _make_gather():
    @functools.partial(
        pl.kernel,
        out_type=jax.ShapeDtypeStruct((N_ROWS, EMB), jnp.float32),
        mesh=plsc.VectorSubcoreMesh(core_axis_name="c", subcore_axis_name="s"),
        compiler_params=pltpu.CompilerParams(use_tc_tiling_on_sc=False),
        scratch_types=[
            pltpu.VMEM((BPW,), jnp.int32),
            pltpu.VMEM((2, CHUNK, EMB), jnp.float32),
            pltpu.SemaphoreType.DMA,
            pltpu.SemaphoreType.DMA,
        ],
    )
    def _gather(idx_hbm, table_hbm, out_hbm, idx_v, rows_v, gsem, osem):
        wid = lax.axis_index("s") * NC + lax.axis_index("c")
        base = wid * BPW
        pltpu.sync_copy(idx_hbm.at[pl.ds(base, BPW)], idx_v)

        def gather_chunk(i, slot):
            return pltpu.async_copy(
                table_hbm.at[idx_v.at[pl.ds(i * CHUNK, CHUNK)]],
                rows_v.at[slot],
                gsem,
            )

        gather_chunk(0, 0).wait()
        for i in range(NCHUNK):
            if i + 1 < NCHUNK:
                nxt = gather_chunk(i + 1, (i + 1) % 2)
            out_cp = pltpu.async_copy(
                rows_v.at[i % 2],
                out_hbm.at[pl.ds(base + i * CHUNK, CHUNK)],
                osem,
            )
            if i + 1 < NCHUNK:
                nxt.wait()
            out_cp.wait()

    return _gather


def _gru_step(x, h, wih, whh, bih, bhh):
    gi = jnp.dot(x, wih, preferred_element_type=jnp.float32) + bih
    gh = jnp.dot(h, whh, preferred_element_type=jnp.float32) + bhh
    r = jax.nn.sigmoid(gi[:, :HID] + gh[:, :HID])
    z = jax.nn.sigmoid(gi[:, HID:2 * HID] + gh[:, HID:2 * HID])
    n = jnp.tanh(gi[:, 2 * HID:] + r * gh[:, 2 * HID:])
    return (1.0 - z) * n + z * h


def _gru_body(x0_ref, x1_ref, wih_ref, whh_ref, bih_ref, bhh_ref,
              out_ref, h_ref):
    t2 = pl.program_id(0)

    @pl.when(t2 == 0)
    def _():
        h_ref[...] = jnp.zeros_like(h_ref)

    wih = wih_ref[...]
    whh = whh_ref[...]
    bih = bih_ref[...]
    bhh = bhh_ref[...]
    h0 = _gru_step(x0_ref[...], h_ref[...], wih, whh, bih, bhh)
    h1 = _gru_step(x1_ref[...], h0, wih, whh, bih, bhh)
    out_ref[:, :HID] = h0
    out_ref[:, HID:] = h1
    h_ref[...] = h1


def _gru(emb2d, wih_t, whh_t, bih, bhh, interpret=False):
    return pl.pallas_call(
        _gru_body,
        grid=(T // 2,),
        in_specs=[
            pl.BlockSpec((B, EMB), lambda t2: (2 * t2, 0)),
            pl.BlockSpec((B, EMB), lambda t2: (2 * t2 + 1, 0)),
            pl.BlockSpec((EMB, 3 * HID), lambda t2: (0, 0)),
            pl.BlockSpec((HID, 3 * HID), lambda t2: (0, 0)),
            pl.BlockSpec((1, 3 * HID), lambda t2: (0, 0)),
            pl.BlockSpec((1, 3 * HID), lambda t2: (0, 0)),
        ],
        out_specs=pl.BlockSpec((B, 2 * HID), lambda t2: (0, t2)),
        out_shape=jax.ShapeDtypeStruct((B, T * HID), jnp.float32),
        scratch_shapes=[pltpu.VMEM((B, HID), jnp.float32)],
        interpret=interpret,
    )(emb2d, emb2d, wih_t, whh_t, bih, bhh)


@jax.jit
def kernel(input, table, W_ih, W_hh, b_ih, b_hh):
    flat_idx = input.T.reshape(-1)
    emb2d = _make_gather()(flat_idx, table)
    out2d = _gru(emb2d, W_ih.T, W_hh.T, b_ih[None], b_hh[None])
    out = out2d.reshape(B, T, HID)
    h_n = out2d[:, (T - 1) * HID:][None]
    return out, h_n


# speedup vs baseline: 7.4431x; 1.0028x over previous
import functools

import jax
import jax.numpy as jnp
from jax import lax
from jax.experimental import pallas as pl
from jax.experimental.pallas import tpu as pltpu
from jax.experimental.pallas import tpu_sc as plsc

NUM_EMB = 1000000
EMB = 32
HID = 64
B = 4096
T = 50

NC = 2
NS = 16
NW = NC * NS

N_ROWS = B * T
BPW = N_ROWS // NW
CHUNK = 800
NCHUNK = BPW // CHUNK
NBUF = 4


@functools.cache
def _make_gather():
    @functools.partial(
        pl.kernel,
        out_type=jax.ShapeDtypeStruct((N_ROWS, EMB), jnp.float32),
        mesh=plsc.VectorSubcoreMesh(core_axis_name="c", subcore_axis_name="s"),
        compiler_params=pltpu.CompilerParams(use_tc_tiling_on_sc=False),
        scratch_types=[
            pltpu.VMEM((BPW,), jnp.int32),
            pltpu.VMEM((NBUF, CHUNK, EMB), jnp.float32),
            pltpu.SemaphoreType.DMA,
            pltpu.SemaphoreType.DMA,
        ],
    )
    def _gather(idx_hbm, table2d, out_hbm, idx_v, rows_v, gsem, osem):
        wid = lax.axis_index("s") * NC + lax.axis_index("c")
        base = wid * BPW
        pltpu.sync_copy(idx_hbm.at[pl.ds(base, BPW)], idx_v)

        def gather_chunk(i):
            return pltpu.async_copy(
                table2d.at[idx_v.at[pl.ds(i * CHUNK, CHUNK)]],
                rows_v.at[i % NBUF],
                gsem,
            )

        copies = [gather_chunk(i) for i in range(NBUF)]
        for i in range(NCHUNK):
            copies[i % NBUF].wait()
            out_cp = pltpu.async_copy(
                rows_v.at[i % NBUF],
                out_hbm.at[pl.ds(base + i * CHUNK, CHUNK)],
                osem,
            )
            if i + NBUF < NCHUNK:
                out_cp.wait()
                copies[i % NBUF] = gather_chunk(i + NBUF)
            else:
                out_cp.wait()

    return _gather


def _gru_step(x, h, wih, whh, bih, bhh):
    gi = jnp.dot(x, wih, preferred_element_type=jnp.float32) + bih
    gh = jnp.dot(h, whh, preferred_element_type=jnp.float32) + bhh
    r = jax.nn.sigmoid(gi[:, :HID] + gh[:, :HID])
    z = jax.nn.sigmoid(gi[:, HID:2 * HID] + gh[:, HID:2 * HID])
    n = jnp.tanh(gi[:, 2 * HID:] + r * gh[:, 2 * HID:])
    return (1.0 - z) * n + z * h


def _gru_body(x0_ref, x1_ref, wih_ref, whh_ref, bih_ref, bhh_ref,
              out_ref, h_ref):
    t2 = pl.program_id(0)

    @pl.when(t2 == 0)
    def _():
        h_ref[...] = jnp.zeros_like(h_ref)

    wih = wih_ref[...]
    whh = whh_ref[...]
    bih = bih_ref[...]
    bhh = bhh_ref[...]
    h0 = _gru_step(x0_ref[...], h_ref[...], wih, whh, bih, bhh)
    h1 = _gru_step(x1_ref[...], h0, wih, whh, bih, bhh)
    out_ref[:, :HID] = h0
    out_ref[:, HID:] = h1
    h_ref[...] = h1


def _gru(emb2d, wih_t, whh_t, bih, bhh, interpret=False):
    return pl.pallas_call(
        _gru_body,
        grid=(T // 2,),
        in_specs=[
            pl.BlockSpec((B, EMB), lambda t2: (2 * t2, 0)),
            pl.BlockSpec((B, EMB), lambda t2: (2 * t2 + 1, 0)),
            pl.BlockSpec((EMB, 3 * HID), lambda t2: (0, 0)),
            pl.BlockSpec((HID, 3 * HID), lambda t2: (0, 0)),
            pl.BlockSpec((1, 3 * HID), lambda t2: (0, 0)),
            pl.BlockSpec((1, 3 * HID), lambda t2: (0, 0)),
        ],
        out_specs=pl.BlockSpec((B, 2 * HID), lambda t2: (0, t2)),
        out_shape=jax.ShapeDtypeStruct((B, T * HID), jnp.float32),
        scratch_shapes=[pltpu.VMEM((B, HID), jnp.float32)],
        interpret=interpret,
    )(emb2d, emb2d, wih_t, whh_t, bih, bhh)


@jax.jit
def kernel(input, table, W_ih, W_hh, b_ih, b_hh):
    flat_idx = input.T.reshape(-1)
    emb2d = _make_gather()(flat_idx, table)
    out2d = _gru(emb2d, W_ih.T, W_hh.T, b_ih[None], b_hh[None])
    out = out2d.reshape(B, T, HID)
    h_n = out2d[:, (T - 1) * HID:][None]
    return out, h_n
